# baseline (speedup 1.0000x reference)
import functools

import jax
import jax.numpy as jnp
from jax import lax
from jax.experimental import pallas as pl
from jax.experimental.pallas import tpu as pltpu
from jax.experimental.pallas import tpu_sc as plsc

B = 16384
K = 64
D = 512
NC, NS = 2, 16
NW = NC * NS
BPW = B // NW
CH = 128
NCH = BPW // CH

_mesh = plsc.VectorSubcoreMesh(core_axis_name="c", subcore_axis_name="s")


@functools.partial(
    pl.kernel,
    out_type=(
        jax.ShapeDtypeStruct((B, K), jnp.float32),
        jax.ShapeDtypeStruct((B, K), jnp.float32),
        jax.ShapeDtypeStruct((B, K), jnp.float32),
        jax.ShapeDtypeStruct((B, D), jnp.float32),
    ),
    mesh=_mesh,
    compiler_params=pltpu.CompilerParams(use_tc_tiling_on_sc=False),
    scratch_types=[
        pltpu.VMEM((BPW,), jnp.int32),
        pltpu.VMEM((BPW,), jnp.int32),
        pltpu.VMEM((CH, K), jnp.float32),
        pltpu.VMEM((CH, K), jnp.float32),
        pltpu.VMEM((CH, K), jnp.float32),
        pltpu.VMEM((CH, D), jnp.float32),
        pltpu.SemaphoreType.DMA,
        pltpu.SemaphoreType.DMA,
        pltpu.SemaphoreType.DMA,
        pltpu.SemaphoreType.DMA,
    ],
)
def _sc_gather(users_hbm, items_hbm, gu_hbm, gi_hbm, tu_hbm, f_hbm,
               gu_out, gi_out, tu_out, fe_out,
               idx_u, idx_i, bu, bi, bt, bf, s0, s1, s2, s3):
    wid = lax.axis_index("s") * NC + lax.axis_index("c")
    base = wid * BPW
    pltpu.sync_copy(users_hbm.at[pl.ds(base, BPW)], idx_u)
    pltpu.sync_copy(items_hbm.at[pl.ds(base, BPW)], idx_i)

    def chunk(c, carry):
        off = base + c * CH
        iu = idx_u.at[pl.ds(c * CH, CH)]
        ii = idx_i.at[pl.ds(c * CH, CH)]
        cu = pltpu.async_copy(gu_hbm.at[iu], bu, s0)
        ci = pltpu.async_copy(gi_hbm.at[ii], bi, s1)
        ct = pltpu.async_copy(tu_hbm.at[iu], bt, s2)
        cf = pltpu.async_copy(f_hbm.at[ii], bf, s3)
        cu.wait()
        pltpu.sync_copy(bu, gu_out.at[pl.ds(off, CH)])
        ci.wait()
        pltpu.sync_copy(bi, gi_out.at[pl.ds(off, CH)])
        ct.wait()
        pltpu.sync_copy(bt, tu_out.at[pl.ds(off, CH)])
        cf.wait()
        pltpu.sync_copy(bf, fe_out.at[pl.ds(off, CH)])
        return carry

    lax.fori_loop(0, NCH, chunk, 0)


RB = 512


def _tc_body(fe_ref, gu_ref, gi_ref, tu_ref, w_ref, b_ref,
             proj_ref, xui_ref):
    proj = jnp.dot(fe_ref[...], w_ref[...],
                   preferred_element_type=jnp.float32) + b_ref[...]
    ss = jnp.sum(proj * proj, axis=1, keepdims=True)
    inv = 1.0 / jnp.maximum(jnp.sqrt(ss), 1e-12)
    pn = proj * inv
    proj_ref[...] = pn
    xui = (jnp.sum(gu_ref[...] * gi_ref[...], axis=1, keepdims=True)
           + jnp.sum(tu_ref[...] * pn, axis=1, keepdims=True))
    xui_ref[...] = xui


def _tc_score(effe_i, gamma_u, gamma_i, theta_u, proj_W, proj_b):
    grid = (B // RB,)
    proj_i, xui = pl.pallas_call(
        _tc_body,
        grid=grid,
        in_specs=[
            pl.BlockSpec((RB, D), lambda i: (i, 0)),
            pl.BlockSpec((RB, K), lambda i: (i, 0)),
            pl.BlockSpec((RB, K), lambda i: (i, 0)),
            pl.BlockSpec((RB, K), lambda i: (i, 0)),
            pl.BlockSpec((D, K), lambda i: (0, 0)),
            pl.BlockSpec((1, K), lambda i: (0, 0)),
        ],
        out_specs=[
            pl.BlockSpec((RB, K), lambda i: (i, 0)),
            pl.BlockSpec((RB, 1), lambda i: (i, 0)),
        ],
        out_shape=[
            jax.ShapeDtypeStruct((B, K), jnp.float32),
            jax.ShapeDtypeStruct((B, 1), jnp.float32),
        ],
    )(effe_i, gamma_u, gamma_i, theta_u, proj_W, proj_b.reshape(1, K))
    return proj_i, xui.reshape(B)


def kernel(users, items, Gu, Gi, Tu, F, proj_W, proj_b):
    gamma_u, gamma_i, theta_u, effe_i = _sc_gather(users, items, Gu, Gi, Tu, F)
    proj_i, xui = _tc_score(effe_i, gamma_u, gamma_i, theta_u, proj_W, proj_b)
    return (xui, gamma_u, gamma_i, theta_u, proj_i)


# speedup vs baseline: 1.4836x; 1.4836x over previous
import functools

import jax
import jax.numpy as jnp
from jax import lax
from jax.experimental import pallas as pl
from jax.experimental.pallas import tpu as pltpu
from jax.experimental.pallas import tpu_sc as plsc

B = 16384
K = 64
D = 512
NC, NS = 2, 16
NW = NC * NS
BPW = B // NW
CH = 128
NCH = BPW // CH
CHF = 64
NCHF = BPW // CHF

_mesh = plsc.VectorSubcoreMesh(core_axis_name="c", subcore_axis_name="s")


@functools.partial(
    pl.kernel,
    out_type=jax.ShapeDtypeStruct((B, D), jnp.float32),
    mesh=_mesh,
    scratch_types=[
        pltpu.VMEM((BPW,), jnp.int32),
        pltpu.VMEM((CHF, D), jnp.float32),
        pltpu.VMEM((CHF, D), jnp.float32),
        pltpu.SemaphoreType.DMA,
        pltpu.SemaphoreType.DMA,
        pltpu.SemaphoreType.DMA,
        pltpu.SemaphoreType.DMA,
    ],
)
def _sc_gather_f(items_hbm, f_hbm, fe_out, idx_i, b0, b1, si0, si1, so0, so1):
    wid = lax.axis_index("s") * NC + lax.axis_index("c")
    base = wid * BPW
    pltpu.sync_copy(items_hbm.at[pl.ds(base, BPW)], idx_i)
    bufs = (b0, b1)
    sin = (si0, si1)
    sout = (so0, so1)

    def issue_in(c):
        return pltpu.async_copy(
            f_hbm.at[idx_i.at[pl.ds(c * CHF, CHF)]], bufs[c & 1], sin[c & 1])

    def issue_out(c):
        return pltpu.async_copy(
            bufs[c & 1], fe_out.at[pl.ds(base + c * CHF, CHF)], sout[c & 1])

    copies_in = {0: issue_in(0), 1: issue_in(1)}
    copies_out = {}
    for c in range(NCHF):
        copies_in[c].wait()
        copies_out[c] = issue_out(c)
        if c + 2 < NCHF:
            copies_out[c].wait()
            copies_in[c + 2] = issue_in(c + 2)
    copies_out[NCHF - 2].wait()
    copies_out[NCHF - 1].wait()


@functools.partial(
    pl.kernel,
    out_type=(
        jax.ShapeDtypeStruct((B, K), jnp.float32),
        jax.ShapeDtypeStruct((B, K), jnp.float32),
        jax.ShapeDtypeStruct((B, K), jnp.float32),
    ),
    mesh=_mesh,
    compiler_params=pltpu.CompilerParams(use_tc_tiling_on_sc=False),
    scratch_types=[
        pltpu.VMEM((BPW,), jnp.int32),
        pltpu.VMEM((BPW,), jnp.int32),
        pltpu.VMEM((CH, K), jnp.float32),
        pltpu.VMEM((CH, K), jnp.float32),
        pltpu.VMEM((CH, K), jnp.float32),
        pltpu.SemaphoreType.DMA,
        pltpu.SemaphoreType.DMA,
        pltpu.SemaphoreType.DMA,
    ],
)
def _sc_gather_emb(users_hbm, items_hbm, gu_hbm, gi_hbm, tu_hbm,
                   gu_out, gi_out, tu_out,
                   idx_u, idx_i, bu, bi, bt, s0, s1, s2):
    wid = lax.axis_index("s") * NC + lax.axis_index("c")
    base = wid * BPW
    pltpu.sync_copy(users_hbm.at[pl.ds(base, BPW)], idx_u)
    pltpu.sync_copy(items_hbm.at[pl.ds(base, BPW)], idx_i)

    def chunk(c, carry):
        off = base + c * CH
        iu = idx_u.at[pl.ds(c * CH, CH)]
        ii = idx_i.at[pl.ds(c * CH, CH)]
        cu = pltpu.async_copy(gu_hbm.at[iu], bu, s0)
        ci = pltpu.async_copy(gi_hbm.at[ii], bi, s1)
        ct = pltpu.async_copy(tu_hbm.at[iu], bt, s2)
        cu.wait()
        pltpu.sync_copy(bu, gu_out.at[pl.ds(off, CH)])
        ci.wait()
        pltpu.sync_copy(bi, gi_out.at[pl.ds(off, CH)])
        ct.wait()
        pltpu.sync_copy(bt, tu_out.at[pl.ds(off, CH)])
        return carry

    lax.fori_loop(0, NCH, chunk, 0)


RB = 512


def _tc_body(fe_ref, gu_ref, gi_ref, tu_ref, w_ref, b_ref,
             proj_ref, xui_ref):
    proj = jnp.dot(fe_ref[...], w_ref[...],
                   preferred_element_type=jnp.float32) + b_ref[...]
    ss = jnp.sum(proj * proj, axis=1, keepdims=True)
    inv = 1.0 / jnp.maximum(jnp.sqrt(ss), 1e-12)
    pn = proj * inv
    proj_ref[...] = pn
    xui = (jnp.sum(gu_ref[...] * gi_ref[...], axis=1, keepdims=True)
           + jnp.sum(tu_ref[...] * pn, axis=1, keepdims=True))
    xui_ref[...] = xui


def _tc_score(effe_i, gamma_u, gamma_i, theta_u, proj_W, proj_b):
    grid = (B // RB,)
    proj_i, xui = pl.pallas_call(
        _tc_body,
        grid=grid,
        in_specs=[
            pl.BlockSpec((RB, D), lambda i: (i, 0)),
            pl.BlockSpec((RB, K), lambda i: (i, 0)),
            pl.BlockSpec((RB, K), lambda i: (i, 0)),
            pl.BlockSpec((RB, K), lambda i: (i, 0)),
            pl.BlockSpec((D, K), lambda i: (0, 0)),
            pl.BlockSpec((1, K), lambda i: (0, 0)),
        ],
        out_specs=[
            pl.BlockSpec((RB, K), lambda i: (i, 0)),
            pl.BlockSpec((RB, 1), lambda i: (i, 0)),
        ],
        out_shape=[
            jax.ShapeDtypeStruct((B, K), jnp.float32),
            jax.ShapeDtypeStruct((B, 1), jnp.float32),
        ],
    )(effe_i, gamma_u, gamma_i, theta_u, proj_W, proj_b.reshape(1, K))
    return proj_i, xui.reshape(B)


def kernel(users, items, Gu, Gi, Tu, F, proj_W, proj_b):
    effe_i = _sc_gather_f(items, F)
    gamma_u, gamma_i, theta_u = _sc_gather_emb(users, items, Gu, Gi, Tu)
    proj_i, xui = _tc_score(effe_i, gamma_u, gamma_i, theta_u, proj_W, proj_b)
    return (xui, gamma_u, gamma_i, theta_u, proj_i)


# speedup vs baseline: 1.6918x; 1.1404x over previous
import functools

import jax
import jax.numpy as jnp
from jax import lax
from jax.experimental import pallas as pl
from jax.experimental.pallas import tpu as pltpu
from jax.experimental.pallas import tpu_sc as plsc

B = 16384
K = 64
D = 512
NC, NS = 2, 16
NW = NC * NS
BPW = B // NW
CHF = 64
NCHF = BPW // CHF
RCH = 16
NRCH = BPW // RCH

_mesh = plsc.VectorSubcoreMesh(core_axis_name="c", subcore_axis_name="s")


@functools.partial(
    pl.kernel,
    out_type=(
        jax.ShapeDtypeStruct((B, K), jnp.float32),
        jax.ShapeDtypeStruct((B, K), jnp.float32),
        jax.ShapeDtypeStruct((B, K), jnp.float32),
        jax.ShapeDtypeStruct((B, D), jnp.float32),
    ),
    mesh=_mesh,
    scratch_types=[
        pltpu.VMEM((BPW,), jnp.int32),
        pltpu.VMEM((BPW,), jnp.int32),
        pltpu.VMEM((CHF, D), jnp.float32),
        pltpu.VMEM((CHF, D), jnp.float32),
        pltpu.VMEM((RCH, K), jnp.float32),
        pltpu.VMEM((RCH, K), jnp.float32),
        pltpu.VMEM((RCH, K), jnp.float32),
        pltpu.SemaphoreType.DMA,
        pltpu.SemaphoreType.DMA,
        pltpu.SemaphoreType.DMA,
        pltpu.SemaphoreType.DMA,
        pltpu.SemaphoreType.DMA,
        pltpu.SemaphoreType.DMA,
        pltpu.SemaphoreType.DMA,
    ],
)
def _sc_gather(users_hbm, items_hbm, gu_hbm, gi_hbm, tu_hbm, f_hbm,
               gu_out, gi_out, tu_out, fe_out,
               idx_u, idx_v, b0, b1, bu, bi, bt,
               si0, si1, so0, so1, g0, g1, g2):
    wid = lax.axis_index("s") * NC + lax.axis_index("c")
    base = wid * BPW
    pltpu.sync_copy(users_hbm.at[pl.ds(base, BPW)], idx_u)
    pltpu.sync_copy(items_hbm.at[pl.ds(base, BPW)], idx_v)
    su = idx_u
    si = idx_v

    bufs = (b0, b1)
    sin = (si0, si1)
    sout = (so0, so1)

    def issue_in(c):
        return pltpu.async_copy(
            f_hbm.at[idx_v.at[pl.ds(c * CHF, CHF)]], bufs[c & 1], sin[c & 1])

    def issue_out(c):
        return pltpu.async_copy(
            bufs[c & 1], fe_out.at[pl.ds(base + c * CHF, CHF)], sout[c & 1])

    copies_in = {0: issue_in(0), 1: issue_in(1)}
    copies_out = {}
    for c in range(NCHF):
        copies_in[c].wait()
        copies_out[c] = issue_out(c)
        if c + 2 < NCHF:
            copies_out[c].wait()
            copies_in[c + 2] = issue_in(c + 2)
    copies_out[NCHF - 2].wait()
    copies_out[NCHF - 1].wait()

    def burst(c, carry):
        j0 = c * RCH
        iu = su[pl.ds(j0, RCH)]
        ii = si[pl.ds(j0, RCH)]
        copies = []
        for r in range(RCH):
            u = iu[r]
            i = ii[r]
            copies.append(pltpu.async_copy(
                gu_hbm.at[pl.ds(u, 1)], bu.at[pl.ds(r, 1)], g0))
            copies.append(pltpu.async_copy(
                tu_hbm.at[pl.ds(u, 1)], bt.at[pl.ds(r, 1)], g1))
            copies.append(pltpu.async_copy(
                gi_hbm.at[pl.ds(i, 1)], bi.at[pl.ds(r, 1)], g2))
        for cp in copies:
            cp.wait()
        o = base + j0
        pltpu.sync_copy(bu, gu_out.at[pl.ds(o, RCH)])
        pltpu.sync_copy(bt, tu_out.at[pl.ds(o, RCH)])
        pltpu.sync_copy(bi, gi_out.at[pl.ds(o, RCH)])
        return carry

    lax.fori_loop(0, NRCH, burst, 0)


RB = 512


def _tc_body(fe_ref, gu_ref, gi_ref, tu_ref, w_ref, b_ref,
             proj_ref, xui_ref):
    proj = jnp.dot(fe_ref[...], w_ref[...],
                   preferred_element_type=jnp.float32) + b_ref[...]
    ss = jnp.sum(proj * proj, axis=1, keepdims=True)
    inv = 1.0 / jnp.maximum(jnp.sqrt(ss), 1e-12)
    pn = proj * inv
    proj_ref[...] = pn
    xui = (jnp.sum(gu_ref[...] * gi_ref[...], axis=1, keepdims=True)
           + jnp.sum(tu_ref[...] * pn, axis=1, keepdims=True))
    xui_ref[...] = xui


def _tc_score(effe_i, gamma_u, gamma_i, theta_u, proj_W, proj_b):
    grid = (B // RB,)
    proj_i, xui = pl.pallas_call(
        _tc_body,
        grid=grid,
        in_specs=[
            pl.BlockSpec((RB, D), lambda i: (i, 0)),
            pl.BlockSpec((RB, K), lambda i: (i, 0)),
            pl.BlockSpec((RB, K), lambda i: (i, 0)),
            pl.BlockSpec((RB, K), lambda i: (i, 0)),
            pl.BlockSpec((D, K), lambda i: (0, 0)),
            pl.BlockSpec((1, K), lambda i: (0, 0)),
        ],
        out_specs=[
            pl.BlockSpec((RB, K), lambda i: (i, 0)),
            pl.BlockSpec((RB, 1), lambda i: (i, 0)),
        ],
        out_shape=[
            jax.ShapeDtypeStruct((B, K), jnp.float32),
            jax.ShapeDtypeStruct((B, 1), jnp.float32),
        ],
    )(effe_i, gamma_u, gamma_i, theta_u, proj_W, proj_b.reshape(1, K))
    return proj_i, xui.reshape(B)


def kernel(users, items, Gu, Gi, Tu, F, proj_W, proj_b):
    gamma_u, gamma_i, theta_u, effe_i = _sc_gather(users, items, Gu, Gi, Tu, F)
    proj_i, xui = _tc_score(effe_i, gamma_u, gamma_i, theta_u, proj_W, proj_b)
    return (xui, gamma_u, gamma_i, theta_u, proj_i)


# speedup vs baseline: 1.8785x; 1.1104x over previous
import functools

import jax
import jax.numpy as jnp
from jax import lax
from jax.experimental import pallas as pl
from jax.experimental.pallas import tpu as pltpu
from jax.experimental.pallas import tpu_sc as plsc

B = 16384
K = 64
D = 512
N = 100000
NC, NS = 2, 16
NW = NC * NS
BPW = B // NW
CHF = 64
NCHF = BPW // CHF
RCH = 16
NRCH = BPW // RCH
TB = 2048

_mesh = plsc.VectorSubcoreMesh(core_axis_name="c", subcore_axis_name="s")


def _tr_body(a_ref, b_ref, c_ref, oa_ref, ob_ref, oc_ref):
    oa_ref[...] = a_ref[...].T
    ob_ref[...] = b_ref[...].T
    oc_ref[...] = c_ref[...].T


def _tc_transpose(gu_t, gi_t, tu_t):
    grid = (pl.cdiv(N, TB),)
    return pl.pallas_call(
        _tr_body,
        grid=grid,
        in_specs=[pl.BlockSpec((K, TB), lambda i: (0, i))] * 3,
        out_specs=[pl.BlockSpec((TB, K), lambda i: (i, 0))] * 3,
        out_shape=[jax.ShapeDtypeStruct((N, K), jnp.float32)] * 3,
    )(gu_t, gi_t, tu_t)


@functools.partial(
    pl.kernel,
    out_type=jax.ShapeDtypeStruct((B, D), jnp.float32),
    mesh=_mesh,
    scratch_types=[
        pltpu.VMEM((BPW,), jnp.int32),
        pltpu.VMEM((CHF, D), jnp.float32),
        pltpu.VMEM((CHF, D), jnp.float32),
        pltpu.SemaphoreType.DMA,
        pltpu.SemaphoreType.DMA,
        pltpu.SemaphoreType.DMA,
        pltpu.SemaphoreType.DMA,
    ],
)
def _sc_gather_f(items_hbm, f_hbm, fe_out, idx_v, b0, b1, si0, si1, so0, so1):
    wid = lax.axis_index("s") * NC + lax.axis_index("c")
    base = wid * BPW
    pltpu.sync_copy(items_hbm.at[pl.ds(base, BPW)], idx_v)
    bufs = (b0, b1)
    sin = (si0, si1)
    sout = (so0, so1)

    def issue_in(c):
        return pltpu.async_copy(
            f_hbm.at[idx_v.at[pl.ds(c * CHF, CHF)]], bufs[c & 1], sin[c & 1])

    def issue_out(c):
        return pltpu.async_copy(
            bufs[c & 1], fe_out.at[pl.ds(base + c * CHF, CHF)], sout[c & 1])

    copies_in = {0: issue_in(0), 1: issue_in(1)}
    copies_out = {}
    for c in range(NCHF):
        copies_in[c].wait()
        copies_out[c] = issue_out(c)
        if c + 2 < NCHF:
            copies_out[c].wait()
            copies_in[c + 2] = issue_in(c + 2)
    copies_out[NCHF - 2].wait()
    copies_out[NCHF - 1].wait()


@functools.partial(
    pl.kernel,
    out_type=(
        jax.ShapeDtypeStruct((B, K), jnp.float32),
        jax.ShapeDtypeStruct((B, K), jnp.float32),
        jax.ShapeDtypeStruct((B, K), jnp.float32),
    ),
    mesh=_mesh,
    scratch_types=[
        pltpu.VMEM((BPW,), jnp.int32),
        pltpu.VMEM((BPW,), jnp.int32),
        pltpu.VMEM((RCH, K), jnp.float32),
        pltpu.VMEM((RCH, K), jnp.float32),
        pltpu.VMEM((RCH, K), jnp.float32),
        pltpu.SemaphoreType.DMA,
        pltpu.SemaphoreType.DMA,
        pltpu.SemaphoreType.DMA,
    ],
)
def _sc_gather_rows(users_hbm, items_hbm, gu_hbm, gi_hbm, tu_hbm,
                    gu_out, gi_out, tu_out,
                    su, si, bu, bi, bt, g0, g1, g2):
    wid = lax.axis_index("s") * NC + lax.axis_index("c")
    base = wid * BPW
    pltpu.sync_copy(users_hbm.at[pl.ds(base, BPW)], su)
    pltpu.sync_copy(items_hbm.at[pl.ds(base, BPW)], si)

    def burst(c, carry):
        j0 = c * RCH
        iu = su[pl.ds(j0, RCH)]
        ii = si[pl.ds(j0, RCH)]
        copies = []
        for r in range(RCH):
            u = iu[r]
            i = ii[r]
            copies.append(pltpu.async_copy(
                gu_hbm.at[pl.ds(u, 1)], bu.at[pl.ds(r, 1)], g0))
            copies.append(pltpu.async_copy(
                tu_hbm.at[pl.ds(u, 1)], bt.at[pl.ds(r, 1)], g1))
            copies.append(pltpu.async_copy(
                gi_hbm.at[pl.ds(i, 1)], bi.at[pl.ds(r, 1)], g2))
        for cp in copies:
            cp.wait()
        o = base + j0
        pltpu.sync_copy(bu, gu_out.at[pl.ds(o, RCH)])
        pltpu.sync_copy(bt, tu_out.at[pl.ds(o, RCH)])
        pltpu.sync_copy(bi, gi_out.at[pl.ds(o, RCH)])
        return carry

    lax.fori_loop(0, NRCH, burst, 0)


RB = 512


def _tc_body(fe_ref, gu_ref, gi_ref, tu_ref, w_ref, b_ref,
             proj_ref, xui_ref):
    proj = jnp.dot(fe_ref[...], w_ref[...],
                   preferred_element_type=jnp.float32) + b_ref[...]
    ss = jnp.sum(proj * proj, axis=1, keepdims=True)
    inv = 1.0 / jnp.maximum(jnp.sqrt(ss), 1e-12)
    pn = proj * inv
    proj_ref[...] = pn
    xui = (jnp.sum(gu_ref[...] * gi_ref[...], axis=1, keepdims=True)
           + jnp.sum(tu_ref[...] * pn, axis=1, keepdims=True))
    xui_ref[...] = xui


def _tc_score(effe_i, gamma_u, gamma_i, theta_u, proj_W, proj_b):
    grid = (B // RB,)
    proj_i, xui = pl.pallas_call(
        _tc_body,
        grid=grid,
        in_specs=[
            pl.BlockSpec((RB, D), lambda i: (i, 0)),
            pl.BlockSpec((RB, K), lambda i: (i, 0)),
            pl.BlockSpec((RB, K), lambda i: (i, 0)),
            pl.BlockSpec((RB, K), lambda i: (i, 0)),
            pl.BlockSpec((D, K), lambda i: (0, 0)),
            pl.BlockSpec((1, K), lambda i: (0, 0)),
        ],
        out_specs=[
            pl.BlockSpec((RB, K), lambda i: (i, 0)),
            pl.BlockSpec((RB, 1), lambda i: (i, 0)),
        ],
        out_shape=[
            jax.ShapeDtypeStruct((B, K), jnp.float32),
            jax.ShapeDtypeStruct((B, 1), jnp.float32),
        ],
    )(effe_i, gamma_u, gamma_i, theta_u, proj_W, proj_b.reshape(1, K))
    return proj_i, xui.reshape(B)


def kernel(users, items, Gu, Gi, Tu, F, proj_W, proj_b):
    gu_rm, gi_rm, tu_rm = _tc_transpose(Gu.T, Gi.T, Tu.T)
    effe_i = _sc_gather_f(items, F)
    gamma_u, gamma_i, theta_u = _sc_gather_rows(users, items, gu_rm, gi_rm, tu_rm)
    proj_i, xui = _tc_score(effe_i, gamma_u, gamma_i, theta_u, proj_W, proj_b)
    return (xui, gamma_u, gamma_i, theta_u, proj_i)
